# Initial kernel scaffold; baseline (speedup 1.0000x reference)
import jax
import jax.numpy as jnp
from jax.experimental import pallas as pl


def kernel(x_nchw, wconv, bconv, wfc, bfc):
    raise NotImplementedError("write your pallas kernel here")


# speedup vs baseline: 1.1579x; 1.1579x over previous
import jax
import jax.numpy as jnp
from jax.experimental import pallas as pl
from jax.experimental.pallas import tpu as pltpu

_BB = 8


def _fwd_kernel(x_ref, wconv_ref, bconv_ref, wfc_ref, bfc_ref, out_ref,
                col_ref):
    bb, hp, wp, cin = x_ref.shape
    h, w = hp - 2, wp - 2
    hw = h * w
    cout = wconv_ref.shape[1]

    for dh in range(3):
        for dw in range(3):
            k = dh * 3 + dw
            col_ref[:, k * cin:(k + 1) * cin] = (
                x_ref[:, dh:dh + h, dw:dw + w, :].reshape(bb * hw, cin))

    feat = jnp.dot(col_ref[...], wconv_ref[...],
                   preferred_element_type=jnp.float32) + bconv_ref[...]
    feat = jnp.maximum(feat, 0.0)
    pooled = jnp.sum(feat.reshape(bb, hw, cout), axis=1)
    out_ref[...] = jnp.dot(pooled, wfc_ref[...],
                           preferred_element_type=jnp.float32) + bfc_ref[...]


def kernel(x_nchw, wconv, bconv, wfc, bfc):
    B, Cin, H, W = x_nchw.shape
    cout = wconv.shape[1]
    num_classes = wfc.shape[1]
    HW = H * W
    Hp, Wp = H + 2, W + 2
    bb = _BB

    x_pad = jnp.pad(jnp.transpose(x_nchw, (0, 2, 3, 1)),
                    ((0, 0), (1, 1), (1, 1), (0, 0))).astype(jnp.bfloat16)

    wconv_b = wconv.astype(jnp.bfloat16)
    bconv_r = bconv.reshape(1, cout)
    wfc_s = (wfc / HW).astype(jnp.float32)
    bfc_r = bfc.reshape(1, num_classes)

    out = pl.pallas_call(
        _fwd_kernel,
        out_shape=jax.ShapeDtypeStruct((B, num_classes), jnp.float32),
        grid_spec=pltpu.PrefetchScalarGridSpec(
            num_scalar_prefetch=0,
            grid=(B // bb,),
            in_specs=[
                pl.BlockSpec((bb, Hp, Wp, Cin), lambda i: (i, 0, 0, 0)),
                pl.BlockSpec((9 * Cin, cout), lambda i: (0, 0)),
                pl.BlockSpec((1, cout), lambda i: (0, 0)),
                pl.BlockSpec((cout, num_classes), lambda i: (0, 0)),
                pl.BlockSpec((1, num_classes), lambda i: (0, 0)),
            ],
            out_specs=pl.BlockSpec((bb, num_classes), lambda i: (i, 0)),
            scratch_shapes=[pltpu.VMEM((bb * HW, 9 * Cin), jnp.bfloat16)],
        ),
        compiler_params=pltpu.CompilerParams(
            dimension_semantics=("parallel",)),
    )(x_pad, wconv_b, bconv_r, wfc_s, bfc_r)

    return out


# speedup vs baseline: 1.1931x; 1.0305x over previous
import jax
import jax.numpy as jnp
from jax import lax
from jax.experimental import pallas as pl
from jax.experimental.pallas import tpu as pltpu

_BB = 8


def _fwd_kernel(x_ref, wall_ref, mask_ref, wfc_ref, bfc_ref, out_ref,
                col_ref):
    bb, cin, _ = x_ref.shape
    gp = mask_ref.shape[0]
    pp = 34 * 34
    cout = wfc_ref.shape[0]

    for b in range(bb):
        base = b * gp
        for dh in range(3):
            for dw in range(3):
                k = dh * 3 + dw
                off = dh * 34 + dw
                col_ref[k * cin:(k + 1) * cin, base:base + pp] = (
                    x_ref[b, :, off:off + pp])
        col_ref[:, base + pp:base + gp] = jnp.zeros(
            (9 * cin + 1, gp - pp), jnp.bfloat16)
    col_ref[9 * cin:9 * cin + 1, :] = jnp.ones(
        (1, bb * gp), jnp.bfloat16)

    feat = lax.dot_general(
        col_ref[...], wall_ref[...],
        (((0,), (0,)), ((), ())),
        preferred_element_type=jnp.float32)
    feat = jnp.maximum(feat, 0.0).reshape(bb, gp, cout) * mask_ref[...]
    pooled = jnp.sum(feat, axis=1)
    out_ref[...] = jnp.dot(pooled, wfc_ref[...],
                           preferred_element_type=jnp.float32) + bfc_ref[...]


def kernel(x_nchw, wconv, bconv, wfc, bfc):
    B, Cin, H, W = x_nchw.shape
    cout = wconv.shape[1]
    num_classes = wfc.shape[1]
    HW = H * W
    Hp, Wp = H + 2, W + 2
    PP = Hp * Wp
    GP = PP + (-PP % 8)
    HALO = 2 * Wp + 2
    bb = _BB

    x_flat = jnp.pad(x_nchw, ((0, 0), (0, 0), (1, 1), (1, 1))).reshape(
        B, Cin, PP)
    x_ext = jnp.pad(x_flat, ((0, 0), (0, 0), (0, HALO))).astype(jnp.bfloat16)

    wall = jnp.concatenate([wconv, bconv.reshape(1, cout)],
                           axis=0).astype(jnp.bfloat16)

    r = jnp.arange(GP)
    p = r + Wp + 1
    pi, pj = p // Wp, p % Wp
    valid = ((p < PP) & (pi >= 1) & (pi < Hp - 1)
             & (pj >= 1) & (pj < Wp - 1))
    mask2d = jnp.broadcast_to(
        valid[:, None], (GP, cout)).astype(jnp.float32)

    wfc_s = (wfc / HW).astype(jnp.float32)
    bfc_r = bfc.reshape(1, num_classes)

    out = pl.pallas_call(
        _fwd_kernel,
        out_shape=jax.ShapeDtypeStruct((B, num_classes), jnp.float32),
        grid_spec=pltpu.PrefetchScalarGridSpec(
            num_scalar_prefetch=0,
            grid=(B // bb,),
            in_specs=[
                pl.BlockSpec((bb, Cin, PP + HALO), lambda i: (i, 0, 0)),
                pl.BlockSpec((9 * Cin + 1, cout), lambda i: (0, 0)),
                pl.BlockSpec((GP, cout), lambda i: (0, 0)),
                pl.BlockSpec((cout, num_classes), lambda i: (0, 0)),
                pl.BlockSpec((1, num_classes), lambda i: (0, 0)),
            ],
            out_specs=pl.BlockSpec((bb, num_classes), lambda i: (i, 0)),
            scratch_shapes=[
                pltpu.VMEM((9 * Cin + 1, bb * GP), jnp.bfloat16)],
        ),
        compiler_params=pltpu.CompilerParams(
            dimension_semantics=("parallel",)),
    )(x_ext, wall, mask2d, wfc_s, bfc_r)

    return out


# speedup vs baseline: 1.8659x; 1.5638x over previous
import jax
import jax.numpy as jnp
from jax import lax
from jax.experimental import pallas as pl
from jax.experimental.pallas import tpu as pltpu

_BB = 8


def _fwd_kernel(x_ref, w3_ref, m0_ref, m2_ref, wfc_ref, bfc_ref, out_ref,
                *scratch):
    bb, cin, hw = x_ref.shape
    w = 32
    kk = 3 * cin + 1
    cout = wfc_ref.shape[0]
    col_refs = scratch[:bb]
    pool_ref = scratch[bb]

    for b in range(bb):
        col = col_refs[b]
        xb = x_ref[b]
        col[cin:2 * cin, :] = xb.astype(jnp.bfloat16)
        col[0:cin, 0:w] = jnp.zeros((cin, w), jnp.bfloat16)
        col[0:cin, w:hw] = xb[:, 0:hw - w].astype(jnp.bfloat16)
        col[2 * cin:3 * cin, 0:hw - w] = xb[:, w:hw].astype(jnp.bfloat16)
        col[2 * cin:3 * cin, hw - w:hw] = jnp.zeros((cin, w), jnp.bfloat16)
        col[kk - 1:kk, :] = jnp.ones((1, hw), jnp.bfloat16)

        u = lax.dot_general(
            col[...], w3_ref[...],
            (((0,), (0,)), ((), ())),
            preferred_element_type=jnp.float32)
        u0 = u[:, 0:cout]
        u1 = u[:, cout:2 * cout]
        u2 = u[:, 2 * cout:3 * cout]
        s0 = jnp.concatenate(
            [jnp.zeros((1, cout), jnp.float32), u0[0:hw - 1]], axis=0)
        s2 = jnp.concatenate(
            [u2[1:hw], jnp.zeros((1, cout), jnp.float32)], axis=0)
        feat = s0 * m0_ref[...] + u1 + s2 * m2_ref[...]
        pool_ref[b:b + 1, :] = jnp.sum(
            jnp.maximum(feat, 0.0), axis=0, keepdims=True)

    out_ref[...] = jnp.dot(pool_ref[...], wfc_ref[...],
                           preferred_element_type=jnp.float32) + bfc_ref[...]


def kernel(x_nchw, wconv, bconv, wfc, bfc):
    B, Cin, H, W = x_nchw.shape
    cout = wconv.shape[1]
    num_classes = wfc.shape[1]
    HW = H * W
    bb = _BB
    KK = 3 * Cin + 1

    x_flat = x_nchw.reshape(B, Cin, HW)

    w3 = jnp.zeros((KK, 3 * cout), jnp.float32)
    for dh in range(3):
        for dw in range(3):
            w3 = w3.at[Cin * dh:Cin * (dh + 1),
                       cout * dw:cout * (dw + 1)].set(
                wconv[Cin * (3 * dh + dw):Cin * (3 * dh + dw + 1), :])
    w3 = w3.at[KK - 1, cout:2 * cout].set(bconv)
    w3 = w3.astype(jnp.bfloat16)

    q = jnp.arange(HW)
    m0 = jnp.broadcast_to(
        (q % W != 0)[:, None], (HW, cout)).astype(jnp.float32)
    m2 = jnp.broadcast_to(
        (q % W != W - 1)[:, None], (HW, cout)).astype(jnp.float32)

    wfc_s = (wfc / HW).astype(jnp.float32)
    bfc_r = bfc.reshape(1, num_classes)

    out = pl.pallas_call(
        _fwd_kernel,
        out_shape=jax.ShapeDtypeStruct((B, num_classes), jnp.float32),
        grid_spec=pltpu.PrefetchScalarGridSpec(
            num_scalar_prefetch=0,
            grid=(B // bb,),
            in_specs=[
                pl.BlockSpec((bb, Cin, HW), lambda i: (i, 0, 0)),
                pl.BlockSpec((KK, 3 * cout), lambda i: (0, 0)),
                pl.BlockSpec((HW, cout), lambda i: (0, 0)),
                pl.BlockSpec((HW, cout), lambda i: (0, 0)),
                pl.BlockSpec((cout, num_classes), lambda i: (0, 0)),
                pl.BlockSpec((1, num_classes), lambda i: (0, 0)),
            ],
            out_specs=pl.BlockSpec((bb, num_classes), lambda i: (i, 0)),
            scratch_shapes=(
                [pltpu.VMEM((KK, HW), jnp.bfloat16) for _ in range(bb)]
                + [pltpu.VMEM((bb, cout), jnp.float32)]),
        ),
        compiler_params=pltpu.CompilerParams(
            dimension_semantics=("parallel",)),
    )(x_flat, w3, m0, m2, wfc_s, bfc_r)

    return out
